# baseline (speedup 1.0000x reference)
import functools

import jax
import jax.numpy as jnp
from jax.experimental import pallas as pl
from jax.experimental.pallas import tpu as pltpu
from jax.experimental.pallas import tpu_sc as plsc

_B = 128
_N = 1024
_D = 384
_K = 512
_GRID = 32

_NC = 2
_NS = 16
_NW = _NC * _NS
_BPW = _B // _NW
_CH = 64
_CPB = _K // _CH
_NCH = _BPW * _CPB
_RING = 4
_DEPTH = 3


def _cumsum_lanes(x):
    c = x
    n = x.shape[1]
    s = 1
    while s < n:
        c = c + jnp.concatenate(
            [jnp.zeros((x.shape[0], s), c.dtype), c[:, : n - s]], axis=1)
        s *= 2
    return c


def _select_body(uv_ref, pos_ref):
    uv = jnp.clip(uv_ref[...], 0.0, 1.0)
    ux = uv[:, 0:1]
    uy = uv[:, 1:2]
    idx = jax.lax.broadcasted_iota(jnp.int32, (_B, _N), 1)
    cx = ((idx % _GRID).astype(jnp.float32) + 0.5) / float(_GRID)
    cy = ((idx // _GRID).astype(jnp.float32) + 0.5) / float(_GRID)
    dx = ux - cx
    dy = uy - cy
    dist = jnp.sqrt(dx * dx + dy * dy)
    bits = jax.lax.bitcast_convert_type(dist, jnp.int32)

    def bs_cond(carry):
        lo, hi = carry
        return jnp.max(hi - lo) > 1

    def bs_step(carry):
        lo, hi = carry
        mid = lo + (hi - lo) // 2
        cnt = jnp.sum((bits <= mid).astype(jnp.int32), axis=1, keepdims=True)
        ge = cnt >= _K
        return jnp.where(ge, lo, mid), jnp.where(ge, mid, hi)

    lo0 = jnp.min(bits, axis=1, keepdims=True) - 1
    hi0 = jnp.max(bits, axis=1, keepdims=True)
    _, t = jax.lax.while_loop(bs_cond, bs_step, (lo0, hi0))
    n_less = jnp.sum((bits < t).astype(jnp.int32), axis=1, keepdims=True)
    m = _K - n_less
    tie = bits == t
    tie_i = tie.astype(jnp.int32)
    tie_excl = _cumsum_lanes(tie_i) - tie_i
    keep = (bits < t) | (tie & (tie_excl < m))
    kc = _cumsum_lanes(keep.astype(jnp.int32))
    pos_ref[...] = jnp.where(keep, kc - 1, (_K - 1) + (idx + 1 - kc))


def _select(pred_uv):
    return pl.pallas_call(
        _select_body,
        out_shape=jax.ShapeDtypeStruct((_B, _N), jnp.int32),
    )(pred_uv)


def _sc_body(pos_hbm, tok_hbm, out_hbm, keep_hbm, rem_hbm,
             pos_v, comb_v, gidx_v, bufs, sems, osems, ksem, rsem):
    c = jax.lax.axis_index("c")
    s = jax.lax.axis_index("s")
    wid = s * _NC + c
    b0 = wid * _BPW
    lane = jax.lax.broadcasted_iota(jnp.int32, (16,), 0)

    pltpu.sync_copy(pos_hbm.at[pl.ds(b0 * _N, _BPW * _N)], pos_v)

    descs = [None] * _RING
    odescs = [None] * _RING

    def fire(k):
        slot = k % _RING
        if odescs[slot] is not None:
            odescs[slot].wait()
            odescs[slot] = None
        descs[slot] = pltpu.async_copy(
            tok_hbm.at[gidx_v.at[pl.ds(k * _CH, _CH)]], bufs[slot], sems[slot])

    def drain(k):
        slot = k % _RING
        descs[slot].wait()
        odescs[slot] = pltpu.async_copy(
            bufs[slot], out_hbm.at[pl.ds(wid * (_BPW * _K) + k * _CH, _CH)],
            osems[slot])

    for q in range(_BPW):
        b = b0 + q

        def compact(i, carry, q=q, b=b):
            p = pos_v[pl.ds(q * _N + i * 16, 16)]
            vals = i * 16 + lane
            mk = p < _K
            cslot = q * _K + jnp.where(mk, p, p + (_BPW - 1) * _K)
            plsc.store_scatter(comb_v, [cslot], vals)
            gslot = jnp.where(mk, q * _K + p, 0)
            plsc.store_scatter(gidx_v, [gslot], vals + b * _N, mask=mk)
            return carry

        jax.lax.fori_loop(0, _N // 16, compact, 0)
        for j in range(_CPB):
            k = q * _CPB + j
            fire(k)
            if k >= _DEPTH:
                drain(k - _DEPTH)

    kd = pltpu.async_copy(
        comb_v.at[pl.ds(0, _BPW * _K)],
        keep_hbm.at[pl.ds(b0 * _K, _BPW * _K)], ksem)
    rd = pltpu.async_copy(
        comb_v.at[pl.ds(_BPW * _K, _BPW * _K)],
        rem_hbm.at[pl.ds(b0 * _K, _BPW * _K)], rsem)

    for k in range(_NCH - _DEPTH, _NCH):
        drain(k)
    for slot in range(_RING):
        if odescs[slot] is not None:
            odescs[slot].wait()
    kd.wait()
    rd.wait()


@functools.cache
def _sc_run():
    return pl.kernel(
        _sc_body,
        out_type=[
            jax.ShapeDtypeStruct((_B * _K, _D), jnp.float32),
            jax.ShapeDtypeStruct((_B * _K,), jnp.int32),
            jax.ShapeDtypeStruct((_B * _K,), jnp.int32),
        ],
        mesh=plsc.VectorSubcoreMesh(
            core_axis_name="c", subcore_axis_name="s",
            num_cores=_NC, num_subcores=_NS),
        scratch_types=[
            pltpu.VMEM((_BPW * _N,), jnp.int32),
            pltpu.VMEM((2 * _BPW * _K,), jnp.int32),
            pltpu.VMEM((_BPW * _K,), jnp.int32),
            [pltpu.VMEM((_CH, _D), jnp.float32) for _ in range(_RING)],
            [pltpu.SemaphoreType.DMA for _ in range(_RING)],
            [pltpu.SemaphoreType.DMA for _ in range(_RING)],
            pltpu.SemaphoreType.DMA,
            pltpu.SemaphoreType.DMA,
        ],
        compiler_params=pltpu.CompilerParams(needs_layout_passes=False),
    )


def kernel(search_tokens, pred_uv):
    pos = _select(pred_uv)
    tok_flat = search_tokens.reshape(_B * _N, _D)
    out_flat, keep_idx, rem_idx = _sc_run()(pos.reshape(-1), tok_flat)
    return (out_flat.reshape(_B, _K, _D), keep_idx.reshape(_B, _K),
            rem_idx.reshape(_B, _K), _K / float(_N))


# speedup vs baseline: 1.0523x; 1.0523x over previous
import functools

import jax
import jax.numpy as jnp
from jax.experimental import pallas as pl
from jax.experimental.pallas import tpu as pltpu
from jax.experimental.pallas import tpu_sc as plsc

_B = 128
_N = 1024
_D = 384
_K = 512
_GRID = 32

_NC = 2
_NS = 16
_NW = _NC * _NS
_BPW = _B // _NW
_CH = 64
_CPB = _K // _CH
_NCH = _BPW * _CPB
_RING = 4
_DEPTH = 3


def _cumsum_lanes(x):
    c = x
    n = x.shape[1]
    s = 1
    while s < n:
        c = c + jnp.concatenate(
            [jnp.zeros((x.shape[0], s), c.dtype), c[:, : n - s]], axis=1)
        s *= 2
    return c


def _select_body(uv_ref, pos_ref):
    uv = jnp.clip(uv_ref[...], 0.0, 1.0)
    ux = uv[:, 0:1]
    uy = uv[:, 1:2]
    idx = jax.lax.broadcasted_iota(jnp.int32, (_B, _N), 1)
    cx = ((idx % _GRID).astype(jnp.float32) + 0.5) / float(_GRID)
    cy = ((idx // _GRID).astype(jnp.float32) + 0.5) / float(_GRID)
    dx = ux - cx
    dy = uy - cy
    dist = jnp.sqrt(dx * dx + dy * dy)
    bits = jax.lax.bitcast_convert_type(dist, jnp.int32)

    def bs_cond(carry):
        lo, hi = carry
        return jnp.max(hi - lo) > 1

    def bs_step(carry):
        lo, hi = carry
        mid = lo + (hi - lo) // 2
        cnt = jnp.sum((bits <= mid).astype(jnp.int32), axis=1, keepdims=True)
        ge = cnt >= _K
        return jnp.where(ge, lo, mid), jnp.where(ge, mid, hi)

    lo0 = jnp.min(bits, axis=1, keepdims=True) - 1
    hi0 = jnp.max(bits, axis=1, keepdims=True)
    _, t = jax.lax.while_loop(bs_cond, bs_step, (lo0, hi0))
    n_less = jnp.sum((bits < t).astype(jnp.int32), axis=1, keepdims=True)
    m = _K - n_less
    tie = bits == t
    tie_i = tie.astype(jnp.int32)
    tie_excl = _cumsum_lanes(tie_i) - tie_i
    keep = (bits < t) | (tie & (tie_excl < m))
    kc = _cumsum_lanes(keep.astype(jnp.int32))
    pos_ref[...] = jnp.where(keep, kc - 1, (_K - 1) + (idx + 1 - kc))


def _select(pred_uv):
    return pl.pallas_call(
        _select_body,
        out_shape=jax.ShapeDtypeStruct((_B, _N), jnp.int32),
    )(pred_uv)


def _sc_body(pos_hbm, tok_hbm, out_hbm, keep_hbm, rem_hbm,
             pos_v, comb_v, gidx_v, bufs, sems, osems, psems, ksem, rsem):
    c = jax.lax.axis_index("c")
    s = jax.lax.axis_index("s")
    wid = s * _NC + c
    b0 = wid * _BPW
    lane = jax.lax.broadcasted_iota(jnp.int32, (16,), 0)

    pdescs = [pltpu.async_copy(pos_hbm.at[b0 + q], pos_v[q], psems[q])
              for q in range(_BPW)]

    descs = [None] * _RING
    odescs = [None] * _RING

    def fire(k):
        slot = k % _RING
        if odescs[slot] is not None:
            odescs[slot].wait()
            odescs[slot] = None
        descs[slot] = pltpu.async_copy(
            tok_hbm.at[gidx_v.at[pl.ds(k * _CH, _CH)]], bufs[slot], sems[slot])

    def drain(k):
        slot = k % _RING
        descs[slot].wait()
        odescs[slot] = pltpu.async_copy(
            bufs[slot], out_hbm.at[pl.ds(wid * (_BPW * _K) + k * _CH, _CH)],
            osems[slot])

    for q in range(_BPW):
        b = b0 + q

        pdescs[q].wait()

        def compact(i, carry, q=q, b=b):
            p = pos_v[q][pl.ds(i * 16, 16)]
            vals = i * 16 + lane
            mk = p < _K
            crow = jnp.where(mk, q, q + _BPW)
            ccol = jnp.where(mk, p, p - _K)
            plsc.store_scatter(comb_v, [crow, ccol], vals)
            gslot = jnp.where(mk, q * _K + p, 0)
            plsc.store_scatter(gidx_v, [gslot], vals + b * _N, mask=mk)
            return carry

        jax.lax.fori_loop(0, _N // 16, compact, 0)
        for j in range(_CPB):
            k = q * _CPB + j
            fire(k)
            if k >= _DEPTH:
                drain(k - _DEPTH)

    kd = pltpu.async_copy(
        comb_v.at[pl.ds(0, _BPW)], keep_hbm.at[pl.ds(b0, _BPW)], ksem)
    rd = pltpu.async_copy(
        comb_v.at[pl.ds(_BPW, _BPW)], rem_hbm.at[pl.ds(b0, _BPW)], rsem)

    for k in range(_NCH - _DEPTH, _NCH):
        drain(k)
    for slot in range(_RING):
        if odescs[slot] is not None:
            odescs[slot].wait()
    kd.wait()
    rd.wait()


@functools.cache
def _sc_run():
    return pl.kernel(
        _sc_body,
        out_type=[
            jax.ShapeDtypeStruct((_B * _K, _D), jnp.float32),
            jax.ShapeDtypeStruct((_B, _K), jnp.int32),
            jax.ShapeDtypeStruct((_B, _K), jnp.int32),
        ],
        mesh=plsc.VectorSubcoreMesh(
            core_axis_name="c", subcore_axis_name="s",
            num_cores=_NC, num_subcores=_NS),
        scratch_types=[
            [pltpu.VMEM((_N,), jnp.int32) for _ in range(_BPW)],
            pltpu.VMEM((2 * _BPW, _K), jnp.int32),
            pltpu.VMEM((_BPW * _K,), jnp.int32),
            [pltpu.VMEM((_CH, _D), jnp.float32) for _ in range(_RING)],
            [pltpu.SemaphoreType.DMA for _ in range(_RING)],
            [pltpu.SemaphoreType.DMA for _ in range(_RING)],
            [pltpu.SemaphoreType.DMA for _ in range(_BPW)],
            pltpu.SemaphoreType.DMA,
            pltpu.SemaphoreType.DMA,
        ],
        compiler_params=pltpu.CompilerParams(needs_layout_passes=False),
    )


def kernel(search_tokens, pred_uv):
    pos = _select(pred_uv)
    tok_flat = search_tokens.reshape(_B * _N, _D)
    out_flat, keep_idx, rem_idx = _sc_run()(pos, tok_flat)
    return (out_flat.reshape(_B, _K, _D), keep_idx, rem_idx, _K / float(_N))


# speedup vs baseline: 1.0773x; 1.0238x over previous
import functools

import jax
import jax.numpy as jnp
from jax.experimental import pallas as pl
from jax.experimental.pallas import tpu as pltpu
from jax.experimental.pallas import tpu_sc as plsc

_B = 128
_N = 1024
_D = 384
_K = 512
_GRID = 32

_NC = 2
_NS = 16
_NW = _NC * _NS
_BPW = _B // _NW
_CH = 64
_CPB = _K // _CH
_NCH = _BPW * _CPB
_RING = 4
_DEPTH = 3


def _cumsum_lanes(x):
    c = x
    n = x.shape[1]
    s = 1
    while s < n:
        c = c + jnp.concatenate(
            [jnp.zeros((x.shape[0], s), c.dtype), c[:, : n - s]], axis=1)
        s *= 2
    return c


def _select_body(uv_ref, pos_ref):
    uv = jnp.clip(uv_ref[...], 0.0, 1.0)
    ux = uv[:, 0:1]
    uy = uv[:, 1:2]
    idx = jax.lax.broadcasted_iota(jnp.int32, (_B, _N), 1)
    cx = ((idx % _GRID).astype(jnp.float32) + 0.5) / float(_GRID)
    cy = ((idx // _GRID).astype(jnp.float32) + 0.5) / float(_GRID)
    dx = ux - cx
    dy = uy - cy
    dist = jnp.sqrt(dx * dx + dy * dy)
    bits = jax.lax.bitcast_convert_type(dist, jnp.int32)

    def bs_step(_, carry):
        lo, hi = carry
        mid = lo + (hi - lo) // 2
        cnt = jnp.sum((bits <= mid).astype(jnp.int32), axis=1, keepdims=True)
        ge = cnt >= _K
        return jnp.where(ge, lo, mid), jnp.where(ge, mid, hi)

    lo0 = jnp.full((_B, 1), 0x39800000 - 1, jnp.int32)
    hi0 = jnp.full((_B, 1), 0x3FC00000, jnp.int32)
    _, t = jax.lax.fori_loop(0, 27, bs_step, (lo0, hi0))
    n_less = jnp.sum((bits < t).astype(jnp.int32), axis=1, keepdims=True)
    m = _K - n_less
    less = bits < t
    tie = bits == t
    packed = less.astype(jnp.int32) + (tie.astype(jnp.int32) << 16)
    pc = _cumsum_lanes(packed)
    tie_cum = pc >> 16
    kc = (pc & 0xFFFF) + jnp.minimum(tie_cum, m)
    keep = less | (tie & (tie_cum <= m))
    pos_ref[...] = jnp.where(keep, kc - 1, (_K - 1) + (idx + 1 - kc))


def _select(pred_uv):
    return pl.pallas_call(
        _select_body,
        out_shape=jax.ShapeDtypeStruct((_B, _N), jnp.int32),
    )(pred_uv)


def _sc_body(pos_hbm, tok_hbm, out_hbm, keep_hbm, rem_hbm,
             pos_v, comb_v, gidx_v, bufs, sems, osems, psems, ksem, rsem):
    c = jax.lax.axis_index("c")
    s = jax.lax.axis_index("s")
    wid = s * _NC + c
    b0 = wid * _BPW
    lane = jax.lax.broadcasted_iota(jnp.int32, (16,), 0)

    pdescs = [pltpu.async_copy(pos_hbm.at[b0 + q], pos_v[q], psems[q])
              for q in range(_BPW)]

    descs = [None] * _RING
    odescs = [None] * _RING

    def fire(k):
        slot = k % _RING
        if odescs[slot] is not None:
            odescs[slot].wait()
            odescs[slot] = None
        descs[slot] = pltpu.async_copy(
            tok_hbm.at[gidx_v.at[pl.ds(k * _CH, _CH)]], bufs[slot], sems[slot])

    def drain(k):
        slot = k % _RING
        descs[slot].wait()
        odescs[slot] = pltpu.async_copy(
            bufs[slot], out_hbm.at[pl.ds(wid * (_BPW * _K) + k * _CH, _CH)],
            osems[slot])

    for q in range(_BPW):
        b = b0 + q

        pdescs[q].wait()

        def compact(i, carry, q=q, b=b):
            p = pos_v[q][pl.ds(i * 16, 16)]
            vals = i * 16 + lane
            mk = p < _K
            crow = jnp.where(mk, q, q + _BPW)
            ccol = jnp.where(mk, p, p - _K)
            plsc.store_scatter(comb_v, [crow, ccol], vals)
            gslot = jnp.where(mk, q * _K + p, 0)
            plsc.store_scatter(gidx_v, [gslot], vals + b * _N, mask=mk)
            return carry

        jax.lax.fori_loop(0, _N // 16, compact, 0)
        for j in range(_CPB):
            k = q * _CPB + j
            fire(k)
            if k >= _DEPTH:
                drain(k - _DEPTH)

    kd = pltpu.async_copy(
        comb_v.at[pl.ds(0, _BPW)], keep_hbm.at[pl.ds(b0, _BPW)], ksem)
    rd = pltpu.async_copy(
        comb_v.at[pl.ds(_BPW, _BPW)], rem_hbm.at[pl.ds(b0, _BPW)], rsem)

    for k in range(_NCH - _DEPTH, _NCH):
        drain(k)
    for slot in range(_RING):
        if odescs[slot] is not None:
            odescs[slot].wait()
    kd.wait()
    rd.wait()


@functools.cache
def _sc_run():
    return pl.kernel(
        _sc_body,
        out_type=[
            jax.ShapeDtypeStruct((_B * _K, _D), jnp.float32),
            jax.ShapeDtypeStruct((_B, _K), jnp.int32),
            jax.ShapeDtypeStruct((_B, _K), jnp.int32),
        ],
        mesh=plsc.VectorSubcoreMesh(
            core_axis_name="c", subcore_axis_name="s",
            num_cores=_NC, num_subcores=_NS),
        scratch_types=[
            [pltpu.VMEM((_N,), jnp.int32) for _ in range(_BPW)],
            pltpu.VMEM((2 * _BPW, _K), jnp.int32),
            pltpu.VMEM((_BPW * _K,), jnp.int32),
            [pltpu.VMEM((_CH, _D), jnp.float32) for _ in range(_RING)],
            [pltpu.SemaphoreType.DMA for _ in range(_RING)],
            [pltpu.SemaphoreType.DMA for _ in range(_RING)],
            [pltpu.SemaphoreType.DMA for _ in range(_BPW)],
            pltpu.SemaphoreType.DMA,
            pltpu.SemaphoreType.DMA,
        ],
        compiler_params=pltpu.CompilerParams(needs_layout_passes=False),
    )


def kernel(search_tokens, pred_uv):
    pos = _select(pred_uv)
    tok_flat = search_tokens.reshape(_B * _N, _D)
    out_flat, keep_idx, rem_idx = _sc_run()(pos, tok_flat)
    return (out_flat.reshape(_B, _K, _D), keep_idx, rem_idx, _K / float(_N))


# speedup vs baseline: 1.0855x; 1.0076x over previous
import functools

import jax
import jax.numpy as jnp
from jax.experimental import pallas as pl
from jax.experimental.pallas import tpu as pltpu
from jax.experimental.pallas import tpu_sc as plsc

_B = 128
_N = 1024
_D = 384
_K = 512
_GRID = 32

_NC = 2
_NS = 16
_NW = _NC * _NS
_BPW = _B // _NW
_CH = 32
_CPB = _K // _CH
_NCH = _BPW * _CPB
_RING = 8
_DEPTH = 6


def _cumsum_lanes(x):
    c = x
    n = x.shape[1]
    s = 1
    while s < n:
        c = c + jnp.concatenate(
            [jnp.zeros((x.shape[0], s), c.dtype), c[:, : n - s]], axis=1)
        s *= 2
    return c


def _select_body(uv_ref, pos_ref):
    uv = jnp.clip(uv_ref[...], 0.0, 1.0)
    ux = uv[:, 0:1]
    uy = uv[:, 1:2]
    idx = jax.lax.broadcasted_iota(jnp.int32, (_B, _N), 1)
    cx = ((idx % _GRID).astype(jnp.float32) + 0.5) / float(_GRID)
    cy = ((idx // _GRID).astype(jnp.float32) + 0.5) / float(_GRID)
    dx = ux - cx
    dy = uy - cy
    dist = jnp.sqrt(dx * dx + dy * dy)
    bits = jax.lax.bitcast_convert_type(dist, jnp.int32)

    def bs_step(_, carry):
        lo, hi = carry
        mid = lo + (hi - lo) // 2
        cnt = jnp.sum((bits <= mid).astype(jnp.int32), axis=1, keepdims=True)
        ge = cnt >= _K
        return jnp.where(ge, lo, mid), jnp.where(ge, mid, hi)

    lo0 = jnp.full((_B, 1), 0x39800000 - 1, jnp.int32)
    hi0 = jnp.full((_B, 1), 0x3FC00000, jnp.int32)
    _, t = jax.lax.fori_loop(0, 27, bs_step, (lo0, hi0))
    n_less = jnp.sum((bits < t).astype(jnp.int32), axis=1, keepdims=True)
    m = _K - n_less
    less = bits < t
    tie = bits == t
    packed = less.astype(jnp.int32) + (tie.astype(jnp.int32) << 16)
    pc = _cumsum_lanes(packed)
    tie_cum = pc >> 16
    kc = (pc & 0xFFFF) + jnp.minimum(tie_cum, m)
    keep = less | (tie & (tie_cum <= m))
    pos_ref[...] = jnp.where(keep, kc - 1, (_K - 1) + (idx + 1 - kc))


def _select(pred_uv):
    return pl.pallas_call(
        _select_body,
        out_shape=jax.ShapeDtypeStruct((_B, _N), jnp.int32),
    )(pred_uv)


def _sc_body(pos_hbm, tok_hbm, out_hbm, keep_hbm, rem_hbm,
             pos_v, comb_v, gidx_v, bufs, sems, osems, psems, ksem, rsem):
    c = jax.lax.axis_index("c")
    s = jax.lax.axis_index("s")
    wid = s * _NC + c
    b0 = wid * _BPW
    lane = jax.lax.broadcasted_iota(jnp.int32, (16,), 0)

    pdescs = [pltpu.async_copy(pos_hbm.at[b0 + q], pos_v[q], psems[q])
              for q in range(_BPW)]

    descs = [None] * _RING
    odescs = [None] * _RING

    def fire(k):
        slot = k % _RING
        if odescs[slot] is not None:
            odescs[slot].wait()
            odescs[slot] = None
        descs[slot] = pltpu.async_copy(
            tok_hbm.at[gidx_v.at[pl.ds(k * _CH, _CH)]], bufs[slot], sems[slot])

    def drain(k):
        slot = k % _RING
        descs[slot].wait()
        odescs[slot] = pltpu.async_copy(
            bufs[slot], out_hbm.at[pl.ds(wid * (_BPW * _K) + k * _CH, _CH)],
            osems[slot])

    for q in range(_BPW):
        b = b0 + q

        pdescs[q].wait()

        def compact(i, carry, q=q, b=b):
            p = pos_v[q][pl.ds(i * 16, 16)]
            vals = i * 16 + lane
            mk = p < _K
            crow = jnp.where(mk, q, q + _BPW)
            ccol = jnp.where(mk, p, p - _K)
            plsc.store_scatter(comb_v, [crow, ccol], vals)
            gslot = jnp.where(mk, q * _K + p, 0)
            plsc.store_scatter(gidx_v, [gslot], vals + b * _N, mask=mk)
            return carry

        jax.lax.fori_loop(0, _N // 16, compact, 0)
        for j in range(_CPB):
            k = q * _CPB + j
            fire(k)
            if k >= _DEPTH:
                drain(k - _DEPTH)

    kd = pltpu.async_copy(
        comb_v.at[pl.ds(0, _BPW)], keep_hbm.at[pl.ds(b0, _BPW)], ksem)
    rd = pltpu.async_copy(
        comb_v.at[pl.ds(_BPW, _BPW)], rem_hbm.at[pl.ds(b0, _BPW)], rsem)

    for k in range(_NCH - _DEPTH, _NCH):
        drain(k)
    for slot in range(_RING):
        if odescs[slot] is not None:
            odescs[slot].wait()
    kd.wait()
    rd.wait()


@functools.cache
def _sc_run():
    return pl.kernel(
        _sc_body,
        out_type=[
            jax.ShapeDtypeStruct((_B * _K, _D), jnp.float32),
            jax.ShapeDtypeStruct((_B, _K), jnp.int32),
            jax.ShapeDtypeStruct((_B, _K), jnp.int32),
        ],
        mesh=plsc.VectorSubcoreMesh(
            core_axis_name="c", subcore_axis_name="s",
            num_cores=_NC, num_subcores=_NS),
        scratch_types=[
            [pltpu.VMEM((_N,), jnp.int32) for _ in range(_BPW)],
            pltpu.VMEM((2 * _BPW, _K), jnp.int32),
            pltpu.VMEM((_BPW * _K,), jnp.int32),
            [pltpu.VMEM((_CH, _D), jnp.float32) for _ in range(_RING)],
            [pltpu.SemaphoreType.DMA for _ in range(_RING)],
            [pltpu.SemaphoreType.DMA for _ in range(_RING)],
            [pltpu.SemaphoreType.DMA for _ in range(_BPW)],
            pltpu.SemaphoreType.DMA,
            pltpu.SemaphoreType.DMA,
        ],
        compiler_params=pltpu.CompilerParams(needs_layout_passes=False),
    )


def kernel(search_tokens, pred_uv):
    pos = _select(pred_uv)
    tok_flat = search_tokens.reshape(_B * _N, _D)
    out_flat, keep_idx, rem_idx = _sc_run()(pos, tok_flat)
    return (out_flat.reshape(_B, _K, _D), keep_idx, rem_idx, _K / float(_N))
